# Initial kernel scaffold; baseline (speedup 1.0000x reference)
import jax
import jax.numpy as jnp
from jax.experimental import pallas as pl


def kernel(p, o):
    raise NotImplementedError("write your pallas kernel here")


# speedup vs baseline: 12.9709x; 12.9709x over previous
import jax
import jax.numpy as jnp
from jax import lax
from jax.experimental import pallas as pl
from jax.experimental.pallas import tpu as pltpu, tpu_sc as plsc

STRIDE = 4
B = 4
N_PER = 8192
NS = N_PER // STRIDE
NCORE = 2
NSUB = 16
TPB = 8
PTS = N_PER // TPB
L = 16
CH = PTS // L

F32 = jnp.float32
I32 = jnp.int32


def _fps_body(xs, ys, zs, np_out, idx_out,
              xv, yv, zv, dv, candv, gflat, tmpv, onp, oidx, spmem):
    c = lax.axis_index("c")
    s = lax.axis_index("s")
    batch = 2 * c + s // TPB
    slot = s % TPB
    base = batch * N_PER + slot * PTS
    bb = batch * N_PER
    grp = (s // TPB) * (TPB * L)

    iota = lax.iota(I32, L)

    pltpu.sync_copy(xs.at[pl.ds(base, PTS)], xv)
    pltpu.sync_copy(ys.at[pl.ds(base, PTS)], yv)
    pltpu.sync_copy(zs.at[pl.ds(base, PTS)], zv)

    big = jnp.full((L,), 1e10, F32)

    def init_chunk(j, _):
        dv[pl.ds(j * L, L)] = big
        return 0

    lax.fori_loop(0, CH, init_chunk, 0)

    def _lane0(v):
        return lax.broadcast(jnp.sum(jnp.where(iota == 0, v, 0.0)), (L,))

    pltpu.sync_copy(xs.at[pl.ds(bb, L)], tmpv)
    lx = _lane0(tmpv[...])
    pltpu.sync_copy(ys.at[pl.ds(bb, L)], tmpv)
    ly = _lane0(tmpv[...])
    pltpu.sync_copy(zs.at[pl.ds(bb, L)], tmpv)
    lz = _lane0(tmpv[...])

    def record(i, gidx_v, px, py, pz):
        pos = jnp.clip(iota + (3 * i - 2), 0, 3 * NS - 1)
        vals = jnp.where(iota == 2, px, jnp.where(iota == 3, py, pz))
        m = (iota >= 2) & (iota <= 4)
        plsc.store_scatter(onp, [pos], vals, mask=m)
        plsc.store_scatter(oidx, [lax.broadcast(i, (L,))], gidx_v,
                           mask=iota == 0)

    record(0, lax.broadcast(bb, (L,)), lx, ly, lz)

    big_i = jnp.full((L,), 2 ** 30, I32)
    neg = jnp.full((L,), -1.0, F32)

    def step(i, carry):
        lx, ly, lz = carry

        def chunk(j, mc):
            m, mj = mc
            off = j * L
            dx = xv[pl.ds(off, L)] - lx
            dy = yv[pl.ds(off, L)] - ly
            dz = zv[pl.ds(off, L)] - lz
            d = dx * dx + dy * dy + dz * dz
            nd = jnp.minimum(dv[pl.ds(off, L)], d)
            dv[pl.ds(off, L)] = nd
            upd = nd > m
            m = jnp.where(upd, nd, m)
            mj = jnp.where(upd, lax.broadcast(j, (L,)), mj)
            return (m, mj)

        m, mj = lax.fori_loop(0, CH, chunk,
                              (neg, jnp.zeros((L,), I32)))
        gm = jnp.max(m)
        gmv = lax.broadcast(gm, (L,))
        lin = mj * L + iota
        li = jnp.min(jnp.where(m == gmv, lin, big_i))
        liv = lax.broadcast(li, (L,))
        cx = plsc.load_gather(xv, [liv])
        cy = plsc.load_gather(yv, [liv])
        cz = plsc.load_gather(zv, [liv])
        gidx_f = lax.broadcast(base + li, (L,)).astype(F32)
        cand = jnp.where(iota == 0, gmv,
               jnp.where(iota == 1, gidx_f,
               jnp.where(iota == 2, cx,
               jnp.where(iota == 3, cy, cz))))
        candv[...] = cand

        boff = (i % 2) * (NSUB * L)
        pltpu.sync_copy(candv, spmem.at[pl.ds(boff + s * L, L)])
        plsc.subcore_barrier()
        pltpu.sync_copy(spmem.at[pl.ds(boff + grp, TPB * L)], gflat)

        l8 = jnp.minimum(iota, TPB - 1) * L
        maxs = plsc.load_gather(gflat, [l8])
        idxf = plsc.load_gather(gflat, [l8 + 1])
        m8 = jnp.where(iota < TPB, maxs, neg)
        gv = lax.broadcast(jnp.max(m8), (L,))
        tie = m8 == gv
        idxi = idxf.astype(I32)
        widx = jnp.min(jnp.where(tie, idxi, big_i))
        widx_v = lax.broadcast(widx, (L,))
        wlane = jnp.min(jnp.where(tie & (idxi == widx_v), iota, big_i))
        wl = lax.broadcast(wlane * L, (L,))
        nlx = plsc.load_gather(gflat, [wl + 2])
        nly = plsc.load_gather(gflat, [wl + 3])
        nlz = plsc.load_gather(gflat, [wl + 4])
        record(i, widx_v, nlx, nly, nlz)
        return (nlx, nly, nlz)

    lax.fori_loop(1, NS, step, (lx, ly, lz))

    @pl.when(slot == 0)
    def _():
        pltpu.sync_copy(onp, np_out.at[pl.ds(batch * (3 * NS), 3 * NS)])
        pltpu.sync_copy(oidx, idx_out.at[pl.ds(batch * NS, NS)])


def _fps(xs, ys, zs):
    mesh = plsc.VectorSubcoreMesh(core_axis_name="c", subcore_axis_name="s",
                                  num_cores=NCORE, num_subcores=NSUB)
    return pl.kernel(
        _fps_body,
        out_type=(jax.ShapeDtypeStruct((B * 3 * NS,), F32),
                  jax.ShapeDtypeStruct((B * NS,), I32)),
        mesh=mesh,
        compiler_params=pltpu.CompilerParams(needs_layout_passes=False),
        scratch_types=(
            pltpu.VMEM((PTS,), F32),
            pltpu.VMEM((PTS,), F32),
            pltpu.VMEM((PTS,), F32),
            pltpu.VMEM((PTS,), F32),
            pltpu.VMEM((L,), F32),
            pltpu.VMEM((TPB * L,), F32),
            pltpu.VMEM((L,), F32),
            pltpu.VMEM((3 * NS,), F32),
            pltpu.VMEM((NS,), I32),
            pltpu.VMEM_SHARED((2 * NSUB * L,), F32),
        ),
    )(xs, ys, zs)


def kernel(p, o):
    xs = p[:, 0]
    ys = p[:, 1]
    zs = p[:, 2]
    np_flat, idx = _fps(xs, ys, zs)
    n_p = np_flat.reshape(B * NS, 3)
    counts = jnp.diff(jnp.concatenate([jnp.zeros((1,), o.dtype), o]))
    n_o = jnp.cumsum(counts // STRIDE).astype(o.dtype)
    return (n_p, n_o, idx)


# speedup vs baseline: 24.6583x; 1.9010x over previous
import jax
import jax.numpy as jnp
from jax import lax
from jax.experimental import pallas as pl
from jax.experimental.pallas import tpu as pltpu, tpu_sc as plsc

STRIDE = 4
B = 4
N_PER = 8192
NS = N_PER // STRIDE
NCORE = 2
NSUB = 16
TPB = 8
PTS = N_PER // TPB
L = 16
CH = PTS // L

F32 = jnp.float32
I32 = jnp.int32


def _fps_body(xs, ys, zs, np_out, idx_out,
              xv, yv, zv, dv, candv, gflat, tmpv, onp, oidx, spmem):
    c = lax.axis_index("c")
    s = lax.axis_index("s")
    batch = 2 * c + s // TPB
    slot = s % TPB
    base = batch * N_PER + slot * PTS
    bb = batch * N_PER
    grp = (s // TPB) * (TPB * L)

    iota = lax.iota(I32, L)

    pltpu.sync_copy(xs.at[pl.ds(base, PTS)], xv)
    pltpu.sync_copy(ys.at[pl.ds(base, PTS)], yv)
    pltpu.sync_copy(zs.at[pl.ds(base, PTS)], zv)

    big = jnp.full((L,), 1e10, F32)

    def init_chunk(j, _):
        dv[pl.ds(j * L, L)] = big
        return 0

    lax.fori_loop(0, CH, init_chunk, 0)

    def _lane0(v):
        return lax.broadcast(jnp.sum(jnp.where(iota == 0, v, 0.0)), (L,))

    pltpu.sync_copy(xs.at[pl.ds(bb, L)], tmpv)
    lx = _lane0(tmpv[...])
    pltpu.sync_copy(ys.at[pl.ds(bb, L)], tmpv)
    ly = _lane0(tmpv[...])
    pltpu.sync_copy(zs.at[pl.ds(bb, L)], tmpv)
    lz = _lane0(tmpv[...])

    def record(i, gidx_v, px, py, pz):
        pos = jnp.clip(iota + (3 * i - 2), 0, 3 * NS - 1)
        vals = jnp.where(iota == 2, px, jnp.where(iota == 3, py, pz))
        m = (iota >= 2) & (iota <= 4)
        plsc.store_scatter(onp, [pos], vals, mask=m)
        plsc.store_scatter(oidx, [lax.broadcast(i, (L,))], gidx_v,
                           mask=iota == 0)

    record(0, lax.broadcast(bb, (L,)), lx, ly, lz)

    big_i = jnp.full((L,), 2 ** 30, I32)
    neg = jnp.full((L,), -1.0, F32)

    def step(i, carry):
        lx, ly, lz = carry

        acc0 = tuple((neg, jnp.zeros((L,), I32)) for _ in range(4))

        @plsc.parallel_loop(0, CH // 4, unroll=2, carry=acc0)
        def chunk(q, acc):
            out = []
            for k in range(4):
                m, mj = acc[k]
                j = q * 4 + k
                off = j * L
                dx = xv[pl.ds(off, L)] - lx
                dy = yv[pl.ds(off, L)] - ly
                dz = zv[pl.ds(off, L)] - lz
                d = dx * dx + dy * dy + dz * dz
                nd = jnp.minimum(dv[pl.ds(off, L)], d)
                dv[pl.ds(off, L)] = nd
                upd = nd > m
                m = jnp.where(upd, nd, m)
                mj = jnp.where(upd, lax.broadcast(j, (L,)), mj)
                out.append((m, mj))
            return tuple(out)

        m, lin = chunk[0][0], chunk[0][1] * L + iota
        for k in range(1, 4):
            mk, link = chunk[k][0], chunk[k][1] * L + iota
            upd = (mk > m) | ((mk == m) & (link < lin))
            m = jnp.where(upd, mk, m)
            lin = jnp.where(upd, link, lin)
        gm = jnp.max(m)
        gmv = lax.broadcast(gm, (L,))
        li = jnp.min(jnp.where(m == gmv, lin, big_i))
        liv = lax.broadcast(li, (L,))
        cx = plsc.load_gather(xv, [liv])
        cy = plsc.load_gather(yv, [liv])
        cz = plsc.load_gather(zv, [liv])
        gidx_f = lax.broadcast(base + li, (L,)).astype(F32)
        cand = jnp.where(iota == 0, gmv,
               jnp.where(iota == 1, gidx_f,
               jnp.where(iota == 2, cx,
               jnp.where(iota == 3, cy, cz))))
        candv[...] = cand

        boff = (i % 2) * (NSUB * L)
        pltpu.sync_copy(candv, spmem.at[pl.ds(boff + s * L, L)])
        plsc.subcore_barrier()
        pltpu.sync_copy(spmem.at[pl.ds(boff + grp, TPB * L)], gflat)

        l8 = jnp.minimum(iota, TPB - 1) * L
        maxs = plsc.load_gather(gflat, [l8])
        idxf = plsc.load_gather(gflat, [l8 + 1])
        m8 = jnp.where(iota < TPB, maxs, neg)
        gv = lax.broadcast(jnp.max(m8), (L,))
        tie = m8 == gv
        idxi = idxf.astype(I32)
        widx = jnp.min(jnp.where(tie, idxi, big_i))
        widx_v = lax.broadcast(widx, (L,))
        wlane = jnp.min(jnp.where(tie & (idxi == widx_v), iota, big_i))
        wl = lax.broadcast(wlane * L, (L,))
        nlx = plsc.load_gather(gflat, [wl + 2])
        nly = plsc.load_gather(gflat, [wl + 3])
        nlz = plsc.load_gather(gflat, [wl + 4])
        record(i, widx_v, nlx, nly, nlz)
        return (nlx, nly, nlz)

    lax.fori_loop(1, NS, step, (lx, ly, lz))

    @pl.when(slot == 0)
    def _():
        pltpu.sync_copy(onp, np_out.at[pl.ds(batch * (3 * NS), 3 * NS)])
        pltpu.sync_copy(oidx, idx_out.at[pl.ds(batch * NS, NS)])


def _fps(xs, ys, zs):
    mesh = plsc.VectorSubcoreMesh(core_axis_name="c", subcore_axis_name="s",
                                  num_cores=NCORE, num_subcores=NSUB)
    return pl.kernel(
        _fps_body,
        out_type=(jax.ShapeDtypeStruct((B * 3 * NS,), F32),
                  jax.ShapeDtypeStruct((B * NS,), I32)),
        mesh=mesh,
        compiler_params=pltpu.CompilerParams(needs_layout_passes=False),
        scratch_types=(
            pltpu.VMEM((PTS,), F32),
            pltpu.VMEM((PTS,), F32),
            pltpu.VMEM((PTS,), F32),
            pltpu.VMEM((PTS,), F32),
            pltpu.VMEM((L,), F32),
            pltpu.VMEM((TPB * L,), F32),
            pltpu.VMEM((L,), F32),
            pltpu.VMEM((3 * NS,), F32),
            pltpu.VMEM((NS,), I32),
            pltpu.VMEM_SHARED((2 * NSUB * L,), F32),
        ),
    )(xs, ys, zs)


def kernel(p, o):
    xs = p[:, 0]
    ys = p[:, 1]
    zs = p[:, 2]
    np_flat, idx = _fps(xs, ys, zs)
    n_p = np_flat.reshape(B * NS, 3)
    counts = jnp.diff(jnp.concatenate([jnp.zeros((1,), o.dtype), o]))
    n_o = jnp.cumsum(counts // STRIDE).astype(o.dtype)
    return (n_p, n_o, idx)


# speedup vs baseline: 24.7613x; 1.0042x over previous
import jax
import jax.numpy as jnp
from jax import lax
from jax.experimental import pallas as pl
from jax.experimental.pallas import tpu as pltpu, tpu_sc as plsc

STRIDE = 4
B = 4
N_PER = 8192
NS = N_PER // STRIDE
NCORE = 2
NSUB = 16
TPB = 8
PTS = N_PER // TPB
L = 16
CH = PTS // L

F32 = jnp.float32
I32 = jnp.int32


def _fps_body(xs, ys, zs, np_out, idx_out,
              xv, yv, zv, dv, candv, gflat, tmpv, onp, oidx, spmem):
    c = lax.axis_index("c")
    s = lax.axis_index("s")
    batch = 2 * c + s // TPB
    slot = s % TPB
    base = batch * N_PER + slot * PTS
    bb = batch * N_PER
    grp = (s // TPB) * (TPB * L)

    iota = lax.iota(I32, L)

    pltpu.sync_copy(xs.at[pl.ds(base, PTS)], xv)
    pltpu.sync_copy(ys.at[pl.ds(base, PTS)], yv)
    pltpu.sync_copy(zs.at[pl.ds(base, PTS)], zv)

    big = jnp.full((L,), 1e10, F32)

    def init_chunk(j, _):
        dv[pl.ds(j * L, L)] = big
        return 0

    lax.fori_loop(0, CH, init_chunk, 0)

    def _lane0(v):
        return lax.broadcast(jnp.sum(jnp.where(iota == 0, v, 0.0)), (L,))

    pltpu.sync_copy(xs.at[pl.ds(bb, L)], tmpv)
    lx = _lane0(tmpv[...])
    pltpu.sync_copy(ys.at[pl.ds(bb, L)], tmpv)
    ly = _lane0(tmpv[...])
    pltpu.sync_copy(zs.at[pl.ds(bb, L)], tmpv)
    lz = _lane0(tmpv[...])

    def record(i, gidx_v, px, py, pz):
        pos = jnp.clip(iota + (3 * i - 2), 0, 3 * NS - 1)
        vals = jnp.where(iota == 2, px, jnp.where(iota == 3, py, pz))
        m = (iota >= 2) & (iota <= 4)
        plsc.store_scatter(onp, [pos], vals, mask=m)
        plsc.store_scatter(oidx, [lax.broadcast(i, (L,))], gidx_v,
                           mask=iota == 0)

    record(0, lax.broadcast(bb, (L,)), lx, ly, lz)

    big_i = jnp.full((L,), 2 ** 30, I32)
    neg = jnp.full((L,), -1.0, F32)

    def step(i, carry):
        lx, ly, lz = carry

        acc0 = tuple((neg, jnp.zeros((L,), I32)) for _ in range(4))

        @plsc.parallel_loop(0, CH // 4, unroll=4, carry=acc0)
        def chunk(q, acc):
            out = []
            for k in range(4):
                m, mj = acc[k]
                j = q * 4 + k
                off = j * L
                dx = xv[pl.ds(off, L)] - lx
                dy = yv[pl.ds(off, L)] - ly
                dz = zv[pl.ds(off, L)] - lz
                d = dx * dx + dy * dy + dz * dz
                nd = jnp.minimum(dv[pl.ds(off, L)], d)
                dv[pl.ds(off, L)] = nd
                upd = nd > m
                m = jnp.where(upd, nd, m)
                mj = jnp.where(upd, lax.broadcast(j, (L,)), mj)
                out.append((m, mj))
            return tuple(out)

        m, lin = chunk[0][0], chunk[0][1] * L + iota
        for k in range(1, 4):
            mk, link = chunk[k][0], chunk[k][1] * L + iota
            upd = (mk > m) | ((mk == m) & (link < lin))
            m = jnp.where(upd, mk, m)
            lin = jnp.where(upd, link, lin)
        gm = jnp.max(m)
        gmv = lax.broadcast(gm, (L,))
        li = jnp.min(jnp.where(m == gmv, lin, big_i))
        liv = lax.broadcast(li, (L,))
        cx = plsc.load_gather(xv, [liv])
        cy = plsc.load_gather(yv, [liv])
        cz = plsc.load_gather(zv, [liv])
        gidx_f = lax.broadcast(base + li, (L,)).astype(F32)
        cand = jnp.where(iota == 0, gmv,
               jnp.where(iota == 1, gidx_f,
               jnp.where(iota == 2, cx,
               jnp.where(iota == 3, cy, cz))))
        candv[...] = cand

        boff = (i % 2) * (NSUB * L)
        pltpu.sync_copy(candv, spmem.at[pl.ds(boff + s * L, L)])
        plsc.subcore_barrier()
        pltpu.sync_copy(spmem.at[pl.ds(boff + grp, TPB * L)], gflat)

        l8 = jnp.minimum(iota, TPB - 1) * L
        maxs = plsc.load_gather(gflat, [l8])
        idxf = plsc.load_gather(gflat, [l8 + 1])
        m8 = jnp.where(iota < TPB, maxs, neg)
        gv = lax.broadcast(jnp.max(m8), (L,))
        tie = m8 == gv
        idxi = idxf.astype(I32)
        widx = jnp.min(jnp.where(tie, idxi, big_i))
        widx_v = lax.broadcast(widx, (L,))
        wlane = jnp.min(jnp.where(tie & (idxi == widx_v), iota, big_i))
        wl = lax.broadcast(wlane * L, (L,))
        nlx = plsc.load_gather(gflat, [wl + 2])
        nly = plsc.load_gather(gflat, [wl + 3])
        nlz = plsc.load_gather(gflat, [wl + 4])
        record(i, widx_v, nlx, nly, nlz)
        return (nlx, nly, nlz)

    lax.fori_loop(1, NS, step, (lx, ly, lz))

    @pl.when(slot == 0)
    def _():
        pltpu.sync_copy(onp, np_out.at[pl.ds(batch * (3 * NS), 3 * NS)])
        pltpu.sync_copy(oidx, idx_out.at[pl.ds(batch * NS, NS)])


def _fps(xs, ys, zs):
    mesh = plsc.VectorSubcoreMesh(core_axis_name="c", subcore_axis_name="s",
                                  num_cores=NCORE, num_subcores=NSUB)
    return pl.kernel(
        _fps_body,
        out_type=(jax.ShapeDtypeStruct((B * 3 * NS,), F32),
                  jax.ShapeDtypeStruct((B * NS,), I32)),
        mesh=mesh,
        compiler_params=pltpu.CompilerParams(needs_layout_passes=False),
        scratch_types=(
            pltpu.VMEM((PTS,), F32),
            pltpu.VMEM((PTS,), F32),
            pltpu.VMEM((PTS,), F32),
            pltpu.VMEM((PTS,), F32),
            pltpu.VMEM((L,), F32),
            pltpu.VMEM((TPB * L,), F32),
            pltpu.VMEM((L,), F32),
            pltpu.VMEM((3 * NS,), F32),
            pltpu.VMEM((NS,), I32),
            pltpu.VMEM_SHARED((2 * NSUB * L,), F32),
        ),
    )(xs, ys, zs)


def kernel(p, o):
    xs = p[:, 0]
    ys = p[:, 1]
    zs = p[:, 2]
    np_flat, idx = _fps(xs, ys, zs)
    n_p = np_flat.reshape(B * NS, 3)
    counts = jnp.diff(jnp.concatenate([jnp.zeros((1,), o.dtype), o]))
    n_o = jnp.cumsum(counts // STRIDE).astype(o.dtype)
    return (n_p, n_o, idx)


# speedup vs baseline: 24.8230x; 1.0025x over previous
import jax
import jax.numpy as jnp
from jax import lax
from jax.experimental import pallas as pl
from jax.experimental.pallas import tpu as pltpu, tpu_sc as plsc

STRIDE = 4
B = 4
N_PER = 8192
NS = N_PER // STRIDE
NCORE = 2
NSUB = 16
TPB = 8
PTS = N_PER // TPB
L = 16
CH = PTS // L

F32 = jnp.float32
I32 = jnp.int32


def _fps_body(xs, ys, zs, np_out, idx_out,
              xv, yv, zv, dv, candv, gflat, tmpv, onp, oidx, spmem):
    c = lax.axis_index("c")
    s = lax.axis_index("s")
    batch = 2 * c + s // TPB
    slot = s % TPB
    base = batch * N_PER + slot * PTS
    bb = batch * N_PER
    grp = (s // TPB) * (TPB * L)

    iota = lax.iota(I32, L)

    pltpu.sync_copy(xs.at[pl.ds(base, PTS)], xv)
    pltpu.sync_copy(ys.at[pl.ds(base, PTS)], yv)
    pltpu.sync_copy(zs.at[pl.ds(base, PTS)], zv)

    big = jnp.full((L,), 1e10, F32)

    def init_chunk(j, _):
        dv[pl.ds(j * L, L)] = big
        return 0

    lax.fori_loop(0, CH, init_chunk, 0)

    def _lane0(v):
        return lax.broadcast(jnp.sum(jnp.where(iota == 0, v, 0.0)), (L,))

    pltpu.sync_copy(xs.at[pl.ds(bb, L)], tmpv)
    lx = _lane0(tmpv[...])
    pltpu.sync_copy(ys.at[pl.ds(bb, L)], tmpv)
    ly = _lane0(tmpv[...])
    pltpu.sync_copy(zs.at[pl.ds(bb, L)], tmpv)
    lz = _lane0(tmpv[...])

    def record(i, gidx_v, px, py, pz):
        pos = jnp.clip(iota + (3 * i - 2), 0, 3 * NS - 1)
        vals = jnp.where(iota == 2, px, jnp.where(iota == 3, py, pz))
        m = (iota >= 2) & (iota <= 4)
        plsc.store_scatter(onp, [pos], vals, mask=m)
        plsc.store_scatter(oidx, [lax.broadcast(i, (L,))], gidx_v,
                           mask=iota == 0)

    record(0, lax.broadcast(bb, (L,)), lx, ly, lz)

    big_i = jnp.full((L,), 2 ** 30, I32)
    neg = jnp.full((L,), -1.0, F32)

    def step(i, carry):
        lx, ly, lz = carry

        acc0 = tuple((neg, jnp.zeros((L,), I32)) for _ in range(4))

        @plsc.parallel_loop(0, CH // 4, unroll=2, carry=acc0)
        def chunk(q, acc):
            out = []
            for k in range(4):
                m, mj = acc[k]
                j = q * 4 + k
                off = j * L
                dx = xv[pl.ds(off, L)] - lx
                dy = yv[pl.ds(off, L)] - ly
                dz = zv[pl.ds(off, L)] - lz
                d = (dx * dx + dz * dz) + dy * dy
                nd = jnp.minimum(dv[pl.ds(off, L)], d)
                dv[pl.ds(off, L)] = nd
                upd = nd > m
                m = jnp.where(upd, nd, m)
                mj = jnp.where(upd, lax.broadcast(j, (L,)), mj)
                out.append((m, mj))
            return tuple(out)

        m, lin = chunk[0][0], chunk[0][1] * L + iota
        for k in range(1, 4):
            mk, link = chunk[k][0], chunk[k][1] * L + iota
            upd = (mk > m) | ((mk == m) & (link < lin))
            m = jnp.where(upd, mk, m)
            lin = jnp.where(upd, link, lin)
        gm = jnp.max(m)
        gmv = lax.broadcast(gm, (L,))
        li = jnp.min(jnp.where(m == gmv, lin, big_i))
        liv = lax.broadcast(li, (L,))
        cx = plsc.load_gather(xv, [liv])
        cy = plsc.load_gather(yv, [liv])
        cz = plsc.load_gather(zv, [liv])
        gidx_f = lax.broadcast(base + li, (L,)).astype(F32)
        cand = jnp.where(iota == 0, gmv,
               jnp.where(iota == 1, gidx_f,
               jnp.where(iota == 2, cx,
               jnp.where(iota == 3, cy, cz))))
        candv[...] = cand

        boff = (i % 2) * (NSUB * L)
        pltpu.sync_copy(candv, spmem.at[pl.ds(boff + s * L, L)])
        plsc.subcore_barrier()
        pltpu.sync_copy(spmem.at[pl.ds(boff + grp, TPB * L)], gflat)

        l8 = jnp.minimum(iota, TPB - 1) * L
        maxs = plsc.load_gather(gflat, [l8])
        idxf = plsc.load_gather(gflat, [l8 + 1])
        m8 = jnp.where(iota < TPB, maxs, neg)
        gv = lax.broadcast(jnp.max(m8), (L,))
        tie = m8 == gv
        idxi = idxf.astype(I32)
        widx = jnp.min(jnp.where(tie, idxi, big_i))
        widx_v = lax.broadcast(widx, (L,))
        wlane = jnp.min(jnp.where(tie & (idxi == widx_v), iota, big_i))
        wl = lax.broadcast(wlane * L, (L,))
        nlx = plsc.load_gather(gflat, [wl + 2])
        nly = plsc.load_gather(gflat, [wl + 3])
        nlz = plsc.load_gather(gflat, [wl + 4])
        record(i, widx_v, nlx, nly, nlz)
        return (nlx, nly, nlz)

    lax.fori_loop(1, NS, step, (lx, ly, lz))

    @pl.when(slot == 0)
    def _():
        pltpu.sync_copy(onp, np_out.at[pl.ds(batch * (3 * NS), 3 * NS)])
        pltpu.sync_copy(oidx, idx_out.at[pl.ds(batch * NS, NS)])


def _fps(xs, ys, zs):
    mesh = plsc.VectorSubcoreMesh(core_axis_name="c", subcore_axis_name="s",
                                  num_cores=NCORE, num_subcores=NSUB)
    return pl.kernel(
        _fps_body,
        out_type=(jax.ShapeDtypeStruct((B * 3 * NS,), F32),
                  jax.ShapeDtypeStruct((B * NS,), I32)),
        mesh=mesh,
        compiler_params=pltpu.CompilerParams(needs_layout_passes=False),
        scratch_types=(
            pltpu.VMEM((PTS,), F32),
            pltpu.VMEM((PTS,), F32),
            pltpu.VMEM((PTS,), F32),
            pltpu.VMEM((PTS,), F32),
            pltpu.VMEM((L,), F32),
            pltpu.VMEM((TPB * L,), F32),
            pltpu.VMEM((L,), F32),
            pltpu.VMEM((3 * NS,), F32),
            pltpu.VMEM((NS,), I32),
            pltpu.VMEM_SHARED((2 * NSUB * L,), F32),
        ),
    )(xs, ys, zs)


def kernel(p, o):
    xs = p[:, 0]
    ys = p[:, 1]
    zs = p[:, 2]
    np_flat, idx = _fps(xs, ys, zs)
    n_p = np_flat.reshape(B * NS, 3)
    counts = jnp.diff(jnp.concatenate([jnp.zeros((1,), o.dtype), o]))
    n_o = jnp.cumsum(counts // STRIDE).astype(o.dtype)
    return (n_p, n_o, idx)
